# baseline (speedup 1.0000x reference)
import functools

import jax
import jax.numpy as jnp
from jax import lax
from jax.experimental import pallas as pl
from jax.experimental.pallas import tpu as pltpu
from jax.experimental.pallas import tpu_sc as plsc

B = 1000
NPG = 100
N = B * NPG
D = 128
H = 128

G = 40
BP = 1024


def _sc_gather(fi_hbm, fv_hbm, idx_hbm, oi_hbm, ov_hbm,
               idx_v, rows_i, rows_v, sem_i, sem_v):
    nc = plsc.get_sparse_core_info().num_cores
    wid = lax.axis_index("s") * nc + lax.axis_index("c")
    bpw = BP // (nc * plsc.get_sparse_core_info().num_subcores)
    base = wid * bpw
    pltpu.sync_copy(idx_hbm.at[pl.ds(base, bpw)], idx_v)
    ci = pltpu.async_copy(fi_hbm.at[idx_v], rows_i, sem_i)
    cv = pltpu.async_copy(fv_hbm.at[idx_v], rows_v, sem_v)
    ci.wait()
    cv.wait()
    pltpu.sync_copy(rows_i, oi_hbm.at[pl.ds(base, bpw)])
    pltpu.sync_copy(rows_v, ov_hbm.at[pl.ds(base, bpw)])


def _gather_last_rows(feat_invar, feat_var, idx_pad):
    info = plsc.get_sparse_core_info()
    bpw = BP // (info.num_cores * info.num_subcores)
    mesh = plsc.VectorSubcoreMesh(core_axis_name="c", subcore_axis_name="s")
    k = functools.partial(
        pl.kernel, mesh=mesh,
        out_type=[jax.ShapeDtypeStruct((BP, D), jnp.float32),
                  jax.ShapeDtypeStruct((BP, D), jnp.float32)],
        scratch_types=[
            pltpu.VMEM((bpw,), jnp.int32),
            pltpu.VMEM((bpw, D), jnp.float32),
            pltpu.VMEM((bpw, D), jnp.float32),
            pltpu.SemaphoreType.DMA,
            pltpu.SemaphoreType.DMA,
        ],
    )(_sc_gather)
    return k(feat_invar, feat_var, idx_pad)


_LOG2E = 1.4426950408889634


def _attn_body(fia_ref, fib_ref, fva_ref, fvb_ref, gvi_ref, gvv_ref,
               wu_ref, bu_ref, wv_ref, we_ref, slog_ref, sbf_ref,
               oi_ref, ov_ref):
    f32 = jnp.float32
    bf = jnp.bfloat16
    dn = (((0,), (0,)), ((), ()))
    Xi_b = jnp.concatenate([fia_ref[...], fib_ref[...]], axis=0).astype(bf)
    Xv_b = jnp.concatenate([fva_ref[...], fvb_ref[...]], axis=0).astype(bf)
    Wu_b = (wu_ref[...] * 0.5).astype(bf)
    bu = bu_ref[...] * 0.5
    Ui = jnp.dot(Xi_b, Wu_b, preferred_element_type=f32)
    Uv = jnp.dot(Xv_b, Wu_b, preferred_element_type=f32)
    Wv_b = (wv_ref[...] * 0.5).astype(bf)
    Vi = jnp.dot(gvi_ref[...].astype(bf), Wv_b,
                 preferred_element_type=f32) + bu
    Vv = jnp.dot(gvv_ref[...].astype(bf), Wv_b,
                 preferred_element_type=f32) + bu
    S_b = sbf_ref[...]
    Vbi = jnp.dot(S_b, Vi.astype(bf), preferred_element_type=f32)
    Vbv = jnp.dot(S_b, Vv.astype(bf), preferred_element_type=f32)
    We_s = we_ref[...] * (0.5 * _LOG2E)
    lane = lax.broadcasted_iota(jnp.int32, (1, 2 * G), 1)
    onesL = (lane < G).astype(f32)
    onesR = 1.0 - onesL
    WeLR = jnp.concatenate([jnp.dot(We_s, onesL), jnp.dot(We_s, onesR)],
                           axis=0).astype(bf)
    T_i = jnp.dot(
        jnp.concatenate([jnp.tanh(Ui + Vbi).astype(bf),
                         jnp.tanh(Ui + Vbv).astype(bf)], axis=1),
        WeLR, preferred_element_type=f32)
    T_v = jnp.dot(
        jnp.concatenate([jnp.tanh(Uv + Vbi).astype(bf),
                         jnp.tanh(Uv + Vbv).astype(bf)], axis=1),
        WeLR, preferred_element_type=f32)
    slog = slog_ref[...]
    A_i_b = jnp.exp2(T_i + slog).astype(bf)
    A_v_b = jnp.exp2(T_v + slog).astype(bf)
    Rp = (lax.dot_general(A_i_b, Xi_b, dn, preferred_element_type=f32) +
          lax.dot_general(A_v_b, Xv_b, dn, preferred_element_type=f32))
    ones_b = jnp.ones((G * NPG, 1), bf)
    sp = (lax.dot_general(A_i_b, ones_b, dn, preferred_element_type=f32) +
          lax.dot_general(A_v_b, ones_b, dn, preferred_element_type=f32))
    oi_ref[0] = Rp[:G] / sp[:G]
    ov_ref[0] = Rp[G:] / sp[G:]


def _attn_readout(fi, fv, gi, gv, Wu, bu2, Wv, We, SLOG, SBF):
    R = G * NPG
    return pl.pallas_call(
        _attn_body,
        grid=(B // G,),
        in_specs=[
            pl.BlockSpec((R // 2, D), lambda i: (2 * i, 0)),
            pl.BlockSpec((R // 2, D), lambda i: (2 * i + 1, 0)),
            pl.BlockSpec((R // 2, D), lambda i: (2 * i, 0)),
            pl.BlockSpec((R // 2, D), lambda i: (2 * i + 1, 0)),
            pl.BlockSpec((G, D), lambda i: (i, 0)),
            pl.BlockSpec((G, D), lambda i: (i, 0)),
            pl.BlockSpec((D, H), lambda i: (0, 0)),
            pl.BlockSpec((1, H), lambda i: (0, 0)),
            pl.BlockSpec((D, H), lambda i: (0, 0)),
            pl.BlockSpec((H, 1), lambda i: (0, 0)),
            pl.BlockSpec((R, 2 * G), lambda i: (0, 0)),
            pl.BlockSpec((R, G), lambda i: (0, 0)),
        ],
        out_specs=[pl.BlockSpec((1, G, D), lambda i: (i, 0, 0)),
                   pl.BlockSpec((1, G, D), lambda i: (i, 0, 0))],
        out_shape=[jax.ShapeDtypeStruct((B // G, G, D), jnp.float32),
                   jax.ShapeDtypeStruct((B // G, G, D), jnp.float32)],
    )(fi, fi, fv, fv, gi, gv, Wu, bu2, Wv, We, SLOG, SBF)


def kernel(feat_invar, feat_var, last_nodes, Wu, bu, Wv, We):
    idx = last_nodes.astype(jnp.int32)
    idx_pad = jnp.concatenate([idx, jnp.zeros((BP - B,), jnp.int32)])
    gi, gv = _gather_last_rows(feat_invar, feat_var, idx_pad)
    bu2 = bu.reshape(1, H)
    S = jnp.repeat(jnp.eye(G, dtype=jnp.float32), NPG, axis=0)
    SLOG = (jnp.concatenate([S, S], axis=1) - 1.0) * 100.0
    SBF = S.astype(jnp.bfloat16)
    ri, rv = _attn_readout(feat_invar, feat_var, gi, gv,
                           Wu, bu2, Wv, We, SLOG, SBF)
    return (ri.reshape(B, D)[:, None, :], rv.reshape(B, D)[:, None, :])
